# Initial kernel scaffold; baseline (speedup 1.0000x reference)
import jax
import jax.numpy as jnp
from jax.experimental import pallas as pl


def kernel(x, cont_feat_mapping, cat_feat_mapping, artist_mapping, artist_emb, W1, b1, W2, b2, cat0, cat1, cat2, cat3, projW, projb, ln_g, ln_b):
    raise NotImplementedError("write your pallas kernel here")


# speedup vs baseline: 2.7177x; 2.7177x over previous
import functools

import jax
import jax.numpy as jnp
from jax import lax
from jax.experimental import pallas as pl
from jax.experimental.pallas import tpu as pltpu
from jax.experimental.pallas import tpu_sc as plsc

NC = 2
NS = 16
NW = NC * NS

B = 1024
T = 200
NTOK = B * T
GR = 4
GTOK = GR * 128
NGRP = NTOK // GTOK
ITERS = -(-NGRP // NW)

N_TRACKS = 1000000
N_CAT = 4
N_CONT = 9
D_ARTIST = 32
D_PER_CAT = 8
D_MODEL = 128
D_CONT = 64
CAT_VOCAB = 1000

CONT_ROWS = N_TRACKS * N_CONT // 16
CAT_ROWS = N_TRACKS * N_CAT // 16
ART_ROWS = N_TRACKS // 16


def _sc_gather_call(x3d, cont16, cat16, art16, artist_emb, c0, c1, c2, c3):
    mesh = plsc.VectorSubcoreMesh(core_axis_name="c", subcore_axis_name="s")

    @functools.partial(
        pl.kernel,
        out_type=(
            jax.ShapeDtypeStruct((NGRP, N_CONT, GTOK), jnp.float32),
            jax.ShapeDtypeStruct((NGRP, GR, 128, D_ARTIST), jnp.float32),
            jax.ShapeDtypeStruct((N_CAT, NGRP, D_PER_CAT, GTOK), jnp.float32),
        ),
        mesh=mesh,
        compiler_params=pltpu.CompilerParams(
            needs_layout_passes=False, use_tc_tiling_on_sc=False),
        scratch_types=[
            pltpu.VMEM((GR, 128), jnp.int32),
            pltpu.VMEM((GR, 128), jnp.int32),
            pltpu.VMEM((GR, 128), jnp.int32),
            pltpu.VMEM((GR, 128), jnp.int32),
            pltpu.VMEM((GR, 128), jnp.int32),
            pltpu.VMEM((GR, 128), jnp.int32),
            pltpu.VMEM((GR, 128, 16), jnp.float32),
            pltpu.VMEM((GR, 128, 16), jnp.float32),
            pltpu.VMEM((GR, 128, 16), jnp.int32),
            pltpu.VMEM((GR, 128, 16), jnp.int32),
            pltpu.VMEM((GR, 128), jnp.int32),
            pltpu.VMEM((N_CAT, CAT_VOCAB, D_PER_CAT), jnp.float32),
            pltpu.VMEM((N_CONT, GTOK), jnp.float32),
            pltpu.VMEM((GR, 128, D_ARTIST), jnp.float32),
            pltpu.VMEM((N_CAT, D_PER_CAT, GTOK), jnp.float32),
            pltpu.SemaphoreType.DMA,
        ],
    )
    def sc_kernel(x_hbm, cont_hbm, cat_hbm, art16_hbm, aemb_hbm,
                  t0, t1, t2, t3,
                  out_cont, out_art, out_cat,
                  xv, r0v, r1v, relv, xq2v, xq4v, contA, contB, catv, artv,
                  artidv, tblv, contT, aembv, cembT, sem):
        wid = lax.axis_index("s") * NC + lax.axis_index("c")
        for c, tbl in enumerate((t0, t1, t2, t3)):
            pltpu.sync_copy(tbl, tblv.at[c])

        def iteration(it, carry):
            gg = it * NW + wid

            @pl.when(gg < NGRP)
            def _():
                pltpu.sync_copy(x_hbm.at[gg], xv)

                def idx_body(j, c2):
                    for k in range(8):
                        sl = pl.ds(k * 16, 16)
                        xx = xv[j, sl]
                        x9 = xx * 9
                        r0 = lax.shift_right_logical(x9, 4)
                        r0v[j, sl] = r0
                        r1v[j, sl] = jnp.minimum(r0 + 1, CONT_ROWS - 1)
                        relv[j, sl] = x9 & 15
                        xq2v[j, sl] = lax.shift_right_logical(xx, 2)
                        xq4v[j, sl] = lax.shift_right_logical(xx, 4)
                    return c2
                lax.fori_loop(0, GR, idx_body, 0)

                cps = []
                for j in range(GR):
                    cps.append(pltpu.async_copy(cont_hbm.at[r0v.at[j]], contA.at[j], sem))
                    cps.append(pltpu.async_copy(cont_hbm.at[r1v.at[j]], contB.at[j], sem))
                    cps.append(pltpu.async_copy(cat_hbm.at[xq2v.at[j]], catv.at[j], sem))
                    cps.append(pltpu.async_copy(art16_hbm.at[xq4v.at[j]], artv.at[j], sem))
                for cp in cps:
                    cp.wait()

                def ext_body(j, c2):
                    jv = jnp.full((16,), j, jnp.int32)
                    for k in range(8):
                        sl = pl.ds(k * 16, 16)
                        rows = lax.iota(jnp.int32, 16) + (k * 16)
                        xx = xv[j, sl]
                        rel = relv[j, sl]
                        tsl = lambda: pl.ds(j * 128 + k * 16, 16)
                        for w in range(N_CONT):
                            rw = rel + w
                            inA = rw < 16
                            vA = plsc.load_gather(
                                contA, [jv, rows, jnp.minimum(rw, 15)])
                            vB = plsc.load_gather(
                                contB, [jv, rows, jnp.maximum(rw - 16, 0)])
                            contT[w, tsl()] = jnp.where(inA, vA, vB)
                        cbase = (xx & 3) * 4
                        for c in range(N_CAT):
                            cv = jnp.full((16,), c, jnp.int32)
                            ids = plsc.load_gather(catv, [jv, rows, cbase + c])
                            for w in range(D_PER_CAT):
                                wv = jnp.full((16,), w, jnp.int32)
                                cembT[c, w, tsl()] = plsc.load_gather(
                                    tblv, [cv, ids, wv])
                        artidv[j, sl] = plsc.load_gather(artv, [jv, rows, xx & 15])
                    return c2
                lax.fori_loop(0, GR, ext_body, 0)

                cps = []
                for j in range(GR):
                    cps.append(pltpu.async_copy(aemb_hbm.at[artidv.at[j]],
                                                aembv.at[j], sem))
                for cp in cps:
                    cp.wait()

                pltpu.sync_copy(contT, out_cont.at[gg])
                pltpu.sync_copy(aembv, out_art.at[gg])
                for c in range(N_CAT):
                    pltpu.sync_copy(cembT.at[c], out_cat.at[c, gg])

            return carry

        lax.fori_loop(0, ITERS, iteration, 0)

    return sc_kernel(x3d, cont16, cat16, art16, artist_emb, c0, c1, c2, c3)


def _tc_dense_body(cont_ref, art_ref, cat_ref, W1_ref, b1_ref, W2_ref, b2_ref,
                   pW_ref, pb_ref, g_ref, bb_ref, out_ref):
    prec = lax.Precision.HIGHEST
    dn_t = (((0,), (0,)), ((), ()))
    cont9 = cont_ref[0]
    h = jnp.maximum(
        lax.dot_general(cont9, W1_ref[...], dn_t, precision=prec,
                        preferred_element_type=jnp.float32) + b1_ref[...], 0.0)
    ec = jnp.dot(h, W2_ref[...], precision=prec,
                 preferred_element_type=jnp.float32) + b2_ref[...]
    pW = pW_ref[...]
    y = jnp.dot(art_ref[...], pW[0:D_ARTIST, :], precision=prec,
                preferred_element_type=jnp.float32)
    y = y + jnp.dot(ec, pW[D_ARTIST:D_ARTIST + D_CONT, :], precision=prec,
                    preferred_element_type=jnp.float32)
    base = D_ARTIST + D_CONT
    for c in range(N_CAT):
        catw = cat_ref[c, 0]
        y = y + lax.dot_general(
            catw, pW[base + c * D_PER_CAT:base + (c + 1) * D_PER_CAT, :], dn_t,
            precision=prec, preferred_element_type=jnp.float32)
    y = y + pb_ref[...]
    mu = jnp.mean(y, axis=-1, keepdims=True)
    d = y - mu
    var = jnp.mean(d * d, axis=-1, keepdims=True)
    out_ref[...] = d * lax.rsqrt(var + 1e-5) * g_ref[...] + bb_ref[...]


def kernel(x, cont_feat_mapping, cat_feat_mapping, artist_mapping, artist_emb,
           W1, b1, W2, b2, cat0, cat1, cat2, cat3, projW, projb, ln_g, ln_b):
    x3d = x.astype(jnp.int32).reshape(NGRP, GR, 128)
    cont16 = cont_feat_mapping.reshape(CONT_ROWS, 16)
    cat16 = cat_feat_mapping.astype(jnp.int32).reshape(CAT_ROWS, 16)
    art16 = artist_mapping.astype(jnp.int32).reshape(ART_ROWS, 16)

    cont_g, art_g, cat_g = _sc_gather_call(
        x3d, cont16, cat16, art16, artist_emb, cat0, cat1, cat2, cat3)

    art_g = art_g.reshape(NTOK, D_ARTIST)

    BT = GTOK
    out = pl.pallas_call(
        _tc_dense_body,
        grid=(NGRP,),
        in_specs=[
            pl.BlockSpec((1, N_CONT, GTOK), lambda i: (i, 0, 0)),
            pl.BlockSpec((BT, D_ARTIST), lambda i: (i, 0)),
            pl.BlockSpec((N_CAT, 1, D_PER_CAT, GTOK), lambda i: (0, i, 0, 0)),
            pl.BlockSpec((N_CONT, D_CONT), lambda i: (0, 0)),
            pl.BlockSpec((1, D_CONT), lambda i: (0, 0)),
            pl.BlockSpec((D_CONT, D_CONT), lambda i: (0, 0)),
            pl.BlockSpec((1, D_CONT), lambda i: (0, 0)),
            pl.BlockSpec((D_MODEL, D_MODEL), lambda i: (0, 0)),
            pl.BlockSpec((1, D_MODEL), lambda i: (0, 0)),
            pl.BlockSpec((1, D_MODEL), lambda i: (0, 0)),
            pl.BlockSpec((1, D_MODEL), lambda i: (0, 0)),
        ],
        out_specs=pl.BlockSpec((BT, D_MODEL), lambda i: (i, 0)),
        out_shape=jax.ShapeDtypeStruct((NTOK, D_MODEL), jnp.float32),
    )(cont_g, art_g, cat_g, W1, b1.reshape(1, -1), W2, b2.reshape(1, -1),
      projW, projb.reshape(1, -1), ln_g.reshape(1, -1), ln_b.reshape(1, -1))

    return out.reshape(B, T, D_MODEL)


# speedup vs baseline: 3.3857x; 1.2458x over previous
import functools

import jax
import jax.numpy as jnp
from jax import lax
from jax.experimental import pallas as pl
from jax.experimental.pallas import tpu as pltpu
from jax.experimental.pallas import tpu_sc as plsc

NC = 2
NS = 16
NW = NC * NS

B = 1024
T = 200
NTOK = B * T
GR = 4
GTOK = GR * 128
NGRP = NTOK // GTOK
ITERS = -(-NGRP // NW)

N_TRACKS = 1000000
N_CAT = 4
N_CONT = 9
D_ARTIST = 32
D_PER_CAT = 8
D_MODEL = 128
D_CONT = 64
CAT_VOCAB = 1000

CONT_ROWS = N_TRACKS * N_CONT // 16
CAT_ROWS = N_TRACKS * N_CAT // 16
ART_ROWS = N_TRACKS // 16


def _sc_gather_call(x3d, cont16, cat16, art16, artist_emb, c0, c1, c2, c3):
    mesh = plsc.VectorSubcoreMesh(core_axis_name="c", subcore_axis_name="s")

    @functools.partial(
        pl.kernel,
        out_type=(
            jax.ShapeDtypeStruct((NGRP, N_CONT, GTOK), jnp.float32),
            jax.ShapeDtypeStruct((NGRP, GR, 128, D_ARTIST), jnp.float32),
            jax.ShapeDtypeStruct((N_CAT, NGRP, D_PER_CAT, GTOK), jnp.float32),
        ),
        mesh=mesh,
        compiler_params=pltpu.CompilerParams(
            needs_layout_passes=False, use_tc_tiling_on_sc=False),
        scratch_types=[
            pltpu.VMEM((GR, 128), jnp.int32),
            pltpu.VMEM((GR, 128), jnp.int32),
            pltpu.VMEM((GR, 128), jnp.int32),
            pltpu.VMEM((GR, 128), jnp.int32),
            pltpu.VMEM((GR, 128), jnp.int32),
            pltpu.VMEM((GR, 128), jnp.int32),
            pltpu.VMEM((GR, 128, 16), jnp.float32),
            pltpu.VMEM((GR, 128, 16), jnp.float32),
            pltpu.VMEM((GR, 128, 16), jnp.int32),
            pltpu.VMEM((GR, 128, 16), jnp.int32),
            pltpu.VMEM((GR, 128), jnp.int32),
            pltpu.VMEM((N_CAT, CAT_VOCAB, D_PER_CAT), jnp.float32),
            pltpu.VMEM((N_CONT, GTOK), jnp.float32),
            pltpu.VMEM((GR, 128, D_ARTIST), jnp.float32),
            pltpu.VMEM((N_CAT, D_PER_CAT, GTOK), jnp.float32),
            pltpu.SemaphoreType.DMA,
        ],
    )
    def sc_kernel(x_hbm, cont_hbm, cat_hbm, art16_hbm, aemb_hbm,
                  t0, t1, t2, t3,
                  out_cont, out_art, out_cat,
                  xv, r0v, r1v, relv, xq2v, xq4v, contA, contB, catv, artv,
                  artidv, tblv, contT, aembv, cembT, sem):
        wid = lax.axis_index("s") * NC + lax.axis_index("c")
        for c, tbl in enumerate((t0, t1, t2, t3)):
            pltpu.sync_copy(tbl, tblv.at[c])

        def iteration(it, carry):
            gg = it * NW + wid

            @pl.when(gg < NGRP)
            def _():
                pltpu.sync_copy(x_hbm.at[gg], xv)

                def idx_body(j, c2):
                    for k in range(8):
                        sl = pl.ds(k * 16, 16)
                        xx = xv[j, sl]
                        x9 = xx * 9
                        r0 = lax.shift_right_logical(x9, 4)
                        r0v[j, sl] = r0
                        r1v[j, sl] = jnp.minimum(r0 + 1, CONT_ROWS - 1)
                        relv[j, sl] = x9 & 15
                        xq2v[j, sl] = lax.shift_right_logical(xx, 2)
                        xq4v[j, sl] = lax.shift_right_logical(xx, 4)
                    return c2
                lax.fori_loop(0, GR, idx_body, 0)

                cps = []
                for j in range(GR):
                    cps.append(pltpu.async_copy(cont_hbm.at[r0v.at[j]], contA.at[j], sem))
                    cps.append(pltpu.async_copy(cont_hbm.at[r1v.at[j]], contB.at[j], sem))
                    cps.append(pltpu.async_copy(cat_hbm.at[xq2v.at[j]], catv.at[j], sem))
                    cps.append(pltpu.async_copy(art16_hbm.at[xq4v.at[j]], artv.at[j], sem))
                for cp in cps:
                    cp.wait()

                def ext_body(j, c2):
                    jv = jnp.full((16,), j, jnp.int32)
                    for k in range(8):
                        sl = pl.ds(k * 16, 16)
                        rows = lax.iota(jnp.int32, 16) + (k * 16)
                        xx = xv[j, sl]
                        rel = relv[j, sl]
                        tsl = lambda: pl.ds(j * 128 + k * 16, 16)
                        for w in range(N_CONT):
                            rw = rel + w
                            inA = rw < 16
                            vA = plsc.load_gather(
                                contA, [jv, rows, jnp.minimum(rw, 15)])
                            vB = plsc.load_gather(
                                contB, [jv, rows, jnp.maximum(rw - 16, 0)])
                            contT[w, tsl()] = jnp.where(inA, vA, vB)
                        cbase = (xx & 3) * 4
                        for c in range(N_CAT):
                            cv = jnp.full((16,), c, jnp.int32)
                            ids = plsc.load_gather(catv, [jv, rows, cbase + c])
                            for w in range(D_PER_CAT):
                                wv = jnp.full((16,), w, jnp.int32)
                                cembT[c, w, tsl()] = plsc.load_gather(
                                    tblv, [cv, ids, wv])
                        artidv[j, sl] = plsc.load_gather(artv, [jv, rows, xx & 15])
                    return c2
                lax.fori_loop(0, GR, ext_body, 0)

                cps = []
                for j in range(GR):
                    cps.append(pltpu.async_copy(aemb_hbm.at[artidv.at[j]],
                                                aembv.at[j], sem))
                for cp in cps:
                    cp.wait()

                pltpu.sync_copy(contT, out_cont.at[gg])
                pltpu.sync_copy(aembv, out_art.at[gg])
                for c in range(N_CAT):
                    pltpu.sync_copy(cembT.at[c], out_cat.at[c, gg])

            return carry

        lax.fori_loop(0, ITERS, iteration, 0)

    return sc_kernel(x3d, cont16, cat16, art16, artist_emb, c0, c1, c2, c3)


def _tr_wide_body(contT_ref, catT_ref, cont_out_ref, cat_out_ref):
    cont_out_ref[...] = contT_ref[...].T
    cat_out_ref[...] = catT_ref[...].T


def _tr_aemb_body(aembT_ref, aemb_out_ref):
    aemb_out_ref[...] = aembT_ref[...].T


def _transpose_tables(cont_feat_mapping, cat_map, artist_emb):
    contT = cont_feat_mapping.T
    catT = cat_map.T
    aembT = artist_emb.T
    W = 8192
    g1 = -(-N_TRACKS // W)
    cont_rm, cat_rm = pl.pallas_call(
        _tr_wide_body,
        grid=(g1,),
        in_specs=[
            pl.BlockSpec((N_CONT, W), lambda i: (0, i)),
            pl.BlockSpec((N_CAT, W), lambda i: (0, i)),
        ],
        out_specs=[
            pl.BlockSpec((W, N_CONT), lambda i: (i, 0)),
            pl.BlockSpec((W, N_CAT), lambda i: (i, 0)),
        ],
        out_shape=[
            jax.ShapeDtypeStruct((N_TRACKS, N_CONT), jnp.float32),
            jax.ShapeDtypeStruct((N_TRACKS, N_CAT), jnp.int32),
        ],
    )(contT, catT)
    g2 = -(-100000 // W)
    aemb_rm = pl.pallas_call(
        _tr_aemb_body,
        grid=(g2,),
        in_specs=[pl.BlockSpec((D_ARTIST, W), lambda i: (0, i))],
        out_specs=pl.BlockSpec((W, D_ARTIST), lambda i: (i, 0)),
        out_shape=jax.ShapeDtypeStruct((100000, D_ARTIST), jnp.float32),
    )(aembT)
    return cont_rm, cat_rm, aemb_rm


def _tc_dense_body(cont_ref, art_ref, cat_ref, W1_ref, b1_ref, W2_ref, b2_ref,
                   pW_ref, pb_ref, g_ref, bb_ref, out_ref):
    prec = lax.Precision.HIGHEST
    dn_t = (((0,), (0,)), ((), ()))
    cont9 = cont_ref[0]
    h = jnp.maximum(
        lax.dot_general(cont9, W1_ref[...], dn_t, precision=prec,
                        preferred_element_type=jnp.float32) + b1_ref[...], 0.0)
    ec = jnp.dot(h, W2_ref[...], precision=prec,
                 preferred_element_type=jnp.float32) + b2_ref[...]
    pW = pW_ref[...]
    y = jnp.dot(art_ref[...], pW[0:D_ARTIST, :], precision=prec,
                preferred_element_type=jnp.float32)
    y = y + jnp.dot(ec, pW[D_ARTIST:D_ARTIST + D_CONT, :], precision=prec,
                    preferred_element_type=jnp.float32)
    base = D_ARTIST + D_CONT
    for c in range(N_CAT):
        catw = cat_ref[c, 0]
        y = y + lax.dot_general(
            catw, pW[base + c * D_PER_CAT:base + (c + 1) * D_PER_CAT, :], dn_t,
            precision=prec, preferred_element_type=jnp.float32)
    y = y + pb_ref[...]
    mu = jnp.mean(y, axis=-1, keepdims=True)
    d = y - mu
    var = jnp.mean(d * d, axis=-1, keepdims=True)
    out_ref[...] = d * lax.rsqrt(var + 1e-5) * g_ref[...] + bb_ref[...]


def kernel(x, cont_feat_mapping, cat_feat_mapping, artist_mapping, artist_emb,
           W1, b1, W2, b2, cat0, cat1, cat2, cat3, projW, projb, ln_g, ln_b):
    x3d = x.astype(jnp.int32).reshape(NGRP, GR, 128)
    cont_rm, cat_rm, aemb_rm = _transpose_tables(
        cont_feat_mapping, cat_feat_mapping.astype(jnp.int32), artist_emb)
    cont16 = cont_rm.reshape(CONT_ROWS, 16)
    cat16 = cat_rm.reshape(CAT_ROWS, 16)
    art16 = artist_mapping.astype(jnp.int32).reshape(ART_ROWS, 16)

    cont_g, art_g, cat_g = _sc_gather_call(
        x3d, cont16, cat16, art16, aemb_rm, cat0, cat1, cat2, cat3)

    art_g = art_g.reshape(NTOK, D_ARTIST)

    BT = GTOK
    out = pl.pallas_call(
        _tc_dense_body,
        grid=(NGRP,),
        in_specs=[
            pl.BlockSpec((1, N_CONT, GTOK), lambda i: (i, 0, 0)),
            pl.BlockSpec((BT, D_ARTIST), lambda i: (i, 0)),
            pl.BlockSpec((N_CAT, 1, D_PER_CAT, GTOK), lambda i: (0, i, 0, 0)),
            pl.BlockSpec((N_CONT, D_CONT), lambda i: (0, 0)),
            pl.BlockSpec((1, D_CONT), lambda i: (0, 0)),
            pl.BlockSpec((D_CONT, D_CONT), lambda i: (0, 0)),
            pl.BlockSpec((1, D_CONT), lambda i: (0, 0)),
            pl.BlockSpec((D_MODEL, D_MODEL), lambda i: (0, 0)),
            pl.BlockSpec((1, D_MODEL), lambda i: (0, 0)),
            pl.BlockSpec((1, D_MODEL), lambda i: (0, 0)),
            pl.BlockSpec((1, D_MODEL), lambda i: (0, 0)),
        ],
        out_specs=pl.BlockSpec((BT, D_MODEL), lambda i: (i, 0)),
        out_shape=jax.ShapeDtypeStruct((NTOK, D_MODEL), jnp.float32),
    )(cont_g, art_g, cat_g, W1, b1.reshape(1, -1), W2, b2.reshape(1, -1),
      projW, projb.reshape(1, -1), ln_g.reshape(1, -1), ln_b.reshape(1, -1))

    return out.reshape(B, T, D_MODEL)


# speedup vs baseline: 5.9926x; 1.7700x over previous
import functools

import jax
import jax.numpy as jnp
from jax import lax
from jax.experimental import pallas as pl
from jax.experimental.pallas import tpu as pltpu
from jax.experimental.pallas import tpu_sc as plsc

NC = 2
NS = 16
NW = NC * NS

B = 1024
T = 200
NTOK = B * T
GR = 4
GTOK = GR * 128
NGRP = NTOK // GTOK
ITERS = -(-NGRP // NW)

N_TRACKS = 1000000
N_CAT = 4
N_CONT = 9
D_ARTIST = 32
D_PER_CAT = 8
D_MODEL = 128
D_CONT = 64
CAT_VOCAB = 1000

CONT_ROWS = N_TRACKS * N_CONT // 16
CAT_ROWS = N_TRACKS * N_CAT // 16
ART_ROWS = N_TRACKS // 16


def _sc_gather_call(x3d, fused, aemb_rm, c0, c1, c2, c3):
    mesh = plsc.VectorSubcoreMesh(core_axis_name="c", subcore_axis_name="s")

    @functools.partial(
        pl.kernel,
        out_type=(
            jax.ShapeDtypeStruct((16, NTOK), jnp.float32),
            jax.ShapeDtypeStruct((NGRP, GR, 128, D_ARTIST), jnp.float32),
            jax.ShapeDtypeStruct((N_CAT, D_PER_CAT, NTOK), jnp.float32),
        ),
        mesh=mesh,
        compiler_params=pltpu.CompilerParams(
            needs_layout_passes=False, use_tc_tiling_on_sc=False),
        scratch_types=[
            pltpu.VMEM((GR, 128), jnp.int32),
            pltpu.VMEM((GR, 128, 16), jnp.float32),
            pltpu.VMEM((GR, 128), jnp.int32),
            pltpu.VMEM((N_CAT, CAT_VOCAB, D_PER_CAT), jnp.float32),
            pltpu.VMEM((N_CONT, GTOK), jnp.float32),
            pltpu.VMEM((GR, 128, D_ARTIST), jnp.float32),
            pltpu.VMEM((N_CAT, D_PER_CAT, GTOK), jnp.float32),
            pltpu.SemaphoreType.DMA,
        ],
    )
    def sc_kernel(x_hbm, fused_hbm, aemb_hbm,
                  t0, t1, t2, t3,
                  out_cont, out_art, out_cat,
                  xv, fblk, artidv, tblv, contT, aembv, cembT, sem):
        wid = lax.axis_index("s") * NC + lax.axis_index("c")
        for c, tbl in enumerate((t0, t1, t2, t3)):
            pltpu.sync_copy(tbl, tblv.at[c])

        def iteration(it, carry):
            gg = it * NW + wid

            @pl.when(gg < NGRP)
            def _():
                pltpu.sync_copy(x_hbm.at[gg], xv)
                cps = []
                for j in range(GR):
                    cps.append(pltpu.async_copy(fused_hbm.at[xv.at[j]],
                                                fblk.at[j], sem))
                for cp in cps:
                    cp.wait()

                def ext_body(j, c2):
                    jv = jnp.full((16,), j, jnp.int32)
                    for k in range(8):
                        sl = pl.ds(k * 16, 16)
                        rows = lax.iota(jnp.int32, 16) + (k * 16)
                        tsl = lambda: pl.ds(j * 128 + k * 16, 16)
                        for w in range(N_CONT):
                            wv = jnp.full((16,), w, jnp.int32)
                            contT[w, tsl()] = plsc.load_gather(
                                fblk, [jv, rows, wv])
                        for c in range(N_CAT):
                            cv = jnp.full((16,), c, jnp.int32)
                            idv = jnp.full((16,), N_CONT + c, jnp.int32)
                            ids = plsc.bitcast(
                                plsc.load_gather(fblk, [jv, rows, idv]),
                                jnp.int32)
                            for w in range(D_PER_CAT):
                                wv = jnp.full((16,), w, jnp.int32)
                                cembT[c, w, tsl()] = plsc.load_gather(
                                    tblv, [cv, ids, wv])
                        a13 = jnp.full((16,), 13, jnp.int32)
                        artidv[j, sl] = plsc.bitcast(
                            plsc.load_gather(fblk, [jv, rows, a13]), jnp.int32)
                    return c2
                lax.fori_loop(0, GR, ext_body, 0)

                cps = []
                for j in range(GR):
                    cps.append(pltpu.async_copy(aemb_hbm.at[artidv.at[j]],
                                                aembv.at[j], sem))
                for cp in cps:
                    cp.wait()

                tok0 = gg * GTOK
                pltpu.sync_copy(contT, out_cont.at[pl.ds(0, N_CONT),
                                                   pl.ds(tok0, GTOK)])
                pltpu.sync_copy(aembv, out_art.at[gg])
                for c in range(N_CAT):
                    pltpu.sync_copy(cembT.at[c],
                                    out_cat.at[c, pl.ds(0, D_PER_CAT),
                                               pl.ds(tok0, GTOK)])

            return carry

        lax.fori_loop(0, ITERS, iteration, 0)

    return sc_kernel(x3d, fused, aemb_rm, c0, c1, c2, c3)


def _tr_fuse_body(contT_ref, catT_ref, art_ref, fused_ref):
    stacked = jnp.concatenate([
        contT_ref[...],
        lax.bitcast_convert_type(catT_ref[...], jnp.float32),
        lax.bitcast_convert_type(art_ref[...], jnp.float32),
    ], axis=0)
    fused_ref[:, 0:14] = stacked.T


def _tr_aemb_body(aembT_ref, aemb_out_ref):
    aemb_out_ref[...] = aembT_ref[...].T


def _transpose_tables(cont_feat_mapping, cat_map, art_map, artist_emb):
    contT = cont_feat_mapping.T
    catT = cat_map.T
    aembT = artist_emb.T
    W = 8192
    g1 = -(-N_TRACKS // W)
    fused = pl.pallas_call(
        _tr_fuse_body,
        grid=(g1,),
        in_specs=[
            pl.BlockSpec((N_CONT, W), lambda i: (0, i)),
            pl.BlockSpec((N_CAT, W), lambda i: (0, i)),
            pl.BlockSpec((1, W), lambda i: (0, i)),
        ],
        out_specs=pl.BlockSpec((W, 16), lambda i: (i, 0)),
        out_shape=jax.ShapeDtypeStruct((N_TRACKS, 16), jnp.float32),
    )(contT, catT, art_map.reshape(1, N_TRACKS))
    g2 = -(-100000 // W)
    aemb_rm = pl.pallas_call(
        _tr_aemb_body,
        grid=(g2,),
        in_specs=[pl.BlockSpec((D_ARTIST, W), lambda i: (0, i))],
        out_specs=pl.BlockSpec((W, D_ARTIST), lambda i: (i, 0)),
        out_shape=jax.ShapeDtypeStruct((100000, D_ARTIST), jnp.float32),
    )(aembT)
    return fused, aemb_rm


GPB = 4
BT = GPB * GTOK


def _tc_dense_body(cont_ref, art_ref, cat_ref, W1_ref, b1_ref, W2_ref, b2_ref,
                   pW_ref, pb_ref, g_ref, bb_ref, out_ref):
    prec = lax.Precision.HIGHEST
    dn_t = (((0,), (0,)), ((), ()))
    pW = pW_ref[...]
    base = D_ARTIST + D_CONT
    P2 = pW[D_ARTIST:base, :]
    W2P = jnp.dot(W2_ref[...], P2, precision=prec,
                  preferred_element_type=jnp.float32)
    bfold = pb_ref[...] + jnp.dot(b2_ref[...], P2, precision=prec,
                                  preferred_element_type=jnp.float32)
    cont9 = cont_ref[0:N_CONT, :]
    h = jnp.maximum(
        lax.dot_general(cont9, W1_ref[...], dn_t, precision=prec,
                        preferred_element_type=jnp.float32) + b1_ref[...], 0.0)
    cat32 = cat_ref[...].reshape(N_CAT * D_PER_CAT, BT)
    y = lax.dot_general(cat32, pW[base:, :], dn_t, precision=prec,
                        preferred_element_type=jnp.float32)
    y = y + jnp.dot(art_ref[...], pW[0:D_ARTIST, :], precision=prec,
                    preferred_element_type=jnp.float32)
    y = y + jnp.dot(h, W2P, precision=prec,
                    preferred_element_type=jnp.float32)
    y = y + bfold
    mu = jnp.mean(y, axis=-1, keepdims=True)
    d = y - mu
    var = jnp.mean(d * d, axis=-1, keepdims=True)
    out_ref[...] = d * lax.rsqrt(var + 1e-5) * g_ref[...] + bb_ref[...]


def kernel(x, cont_feat_mapping, cat_feat_mapping, artist_mapping, artist_emb,
           W1, b1, W2, b2, cat0, cat1, cat2, cat3, projW, projb, ln_g, ln_b):
    x3d = x.astype(jnp.int32).reshape(NGRP, GR, 128)
    fused, aemb_rm = _transpose_tables(
        cont_feat_mapping, cat_feat_mapping.astype(jnp.int32),
        artist_mapping.astype(jnp.int32), artist_emb)

    cont_g, art_g, cat_g = _sc_gather_call(
        x3d, fused, aemb_rm, cat0, cat1, cat2, cat3)

    art_g = art_g.reshape(NTOK, D_ARTIST)

    out = pl.pallas_call(
        _tc_dense_body,
        grid=(NTOK // BT,),
        in_specs=[
            pl.BlockSpec((16, BT), lambda i: (0, i)),
            pl.BlockSpec((BT, D_ARTIST), lambda i: (i, 0)),
            pl.BlockSpec((N_CAT, D_PER_CAT, BT), lambda i: (0, 0, i)),
            pl.BlockSpec((N_CONT, D_CONT), lambda i: (0, 0)),
            pl.BlockSpec((1, D_CONT), lambda i: (0, 0)),
            pl.BlockSpec((D_CONT, D_CONT), lambda i: (0, 0)),
            pl.BlockSpec((1, D_CONT), lambda i: (0, 0)),
            pl.BlockSpec((D_MODEL, D_MODEL), lambda i: (0, 0)),
            pl.BlockSpec((1, D_MODEL), lambda i: (0, 0)),
            pl.BlockSpec((1, D_MODEL), lambda i: (0, 0)),
            pl.BlockSpec((1, D_MODEL), lambda i: (0, 0)),
        ],
        out_specs=pl.BlockSpec((BT, D_MODEL), lambda i: (i, 0)),
        out_shape=jax.ShapeDtypeStruct((NTOK, D_MODEL), jnp.float32),
        compiler_params=pltpu.CompilerParams(
            fuse_transposed_lhs_in_matmul=True),
    )(cont_g, art_g, cat_g, W1, b1.reshape(1, -1), W2, b2.reshape(1, -1),
      projW, projb.reshape(1, -1), ln_g.reshape(1, -1), ln_b.reshape(1, -1))

    return out.reshape(B, T, D_MODEL)


# speedup vs baseline: 7.3863x; 1.2326x over previous
import functools

import jax
import jax.numpy as jnp
from jax import lax
from jax.experimental import pallas as pl
from jax.experimental.pallas import tpu as pltpu
from jax.experimental.pallas import tpu_sc as plsc

NC = 2
NS = 16
NW = NC * NS

B = 1024
T = 200
NTOK = B * T
GR = 4
GTOK = GR * 128
NGRP = NTOK // GTOK
ITERS = -(-NGRP // NW)

N_TRACKS = 1000000
N_CAT = 4
N_CONT = 9
D_ARTIST = 32
D_PER_CAT = 8
D_MODEL = 128
D_CONT = 64
CAT_VOCAB = 1000

CONT_ROWS = N_TRACKS * N_CONT // 16
CAT_ROWS = N_TRACKS * N_CAT // 16
ART_ROWS = N_TRACKS // 16


def _sc_gather_call(x3d, fused, aemb_rm, c0, c1, c2, c3):
    mesh = plsc.VectorSubcoreMesh(core_axis_name="c", subcore_axis_name="s")

    @functools.partial(
        pl.kernel,
        out_type=(
            jax.ShapeDtypeStruct((16, NTOK), jnp.float32),
            jax.ShapeDtypeStruct((NGRP, GR, 128, D_ARTIST), jnp.float32),
            jax.ShapeDtypeStruct((N_CAT, D_PER_CAT, NTOK), jnp.float32),
        ),
        mesh=mesh,
        compiler_params=pltpu.CompilerParams(
            needs_layout_passes=False, use_tc_tiling_on_sc=False),
        scratch_types=[
            pltpu.VMEM((GR, 128), jnp.int32),
            pltpu.VMEM((GR, 128, 16), jnp.float32),
            pltpu.VMEM((GR, 128), jnp.int32),
            pltpu.VMEM((N_CAT, CAT_VOCAB, D_PER_CAT), jnp.float32),
            pltpu.VMEM((N_CONT, GTOK), jnp.float32),
            pltpu.VMEM((GR, 128, D_ARTIST), jnp.float32),
            pltpu.VMEM((N_CAT, D_PER_CAT, GTOK), jnp.float32),
            pltpu.SemaphoreType.DMA,
        ],
    )
    def sc_kernel(x_hbm, fused_hbm, aemb_hbm,
                  t0, t1, t2, t3,
                  out_cont, out_art, out_cat,
                  xv, fblk, artidv, tblv, contT, aembv, cembT, sem):
        wid = lax.axis_index("s") * NC + lax.axis_index("c")
        for c, tbl in enumerate((t0, t1, t2, t3)):
            pltpu.sync_copy(tbl, tblv.at[c])

        def iteration(it, carry):
            gg = it * NW + wid

            @pl.when(gg < NGRP)
            def _():
                pltpu.sync_copy(x_hbm.at[gg], xv)
                cps = []
                for j in range(GR):
                    cps.append(pltpu.async_copy(fused_hbm.at[xv.at[j]],
                                                fblk.at[j], sem))
                for cp in cps:
                    cp.wait()

                def ext_body(j, c2):
                    jv = jnp.full((16,), j, jnp.int32)
                    for k in range(8):
                        sl = pl.ds(k * 16, 16)
                        rows = lax.iota(jnp.int32, 16) + (k * 16)
                        tsl = lambda: pl.ds(j * 128 + k * 16, 16)
                        for w in range(N_CONT):
                            wv = jnp.full((16,), w, jnp.int32)
                            contT[w, tsl()] = plsc.load_gather(
                                fblk, [jv, rows, wv])
                        for c in range(N_CAT):
                            cv = jnp.full((16,), c, jnp.int32)
                            idv = jnp.full((16,), N_CONT + c, jnp.int32)
                            ids = plsc.bitcast(
                                plsc.load_gather(fblk, [jv, rows, idv]),
                                jnp.int32)
                            for w in range(D_PER_CAT):
                                wv = jnp.full((16,), w, jnp.int32)
                                cembT[c, w, tsl()] = plsc.load_gather(
                                    tblv, [cv, ids, wv])
                        a13 = jnp.full((16,), 13, jnp.int32)
                        artidv[j, sl] = plsc.bitcast(
                            plsc.load_gather(fblk, [jv, rows, a13]), jnp.int32)
                    return c2
                lax.fori_loop(0, GR, ext_body, 0)

                cps = []
                for j in range(GR):
                    cps.append(pltpu.async_copy(aemb_hbm.at[artidv.at[j]],
                                                aembv.at[j], sem))
                for cp in cps:
                    cp.wait()

                tok0 = gg * GTOK
                pltpu.sync_copy(contT, out_cont.at[pl.ds(0, N_CONT),
                                                   pl.ds(tok0, GTOK)])
                pltpu.sync_copy(aembv, out_art.at[gg])
                for c in range(N_CAT):
                    pltpu.sync_copy(cembT.at[c],
                                    out_cat.at[c, pl.ds(0, D_PER_CAT),
                                               pl.ds(tok0, GTOK)])

            return carry

        lax.fori_loop(0, ITERS, iteration, 0)

    return sc_kernel(x3d, fused, aemb_rm, c0, c1, c2, c3)


def _tr_fuse_body(contT_ref, catT_ref, art_ref, fused_ref):
    stacked = jnp.concatenate([
        contT_ref[...],
        lax.bitcast_convert_type(catT_ref[...], jnp.float32),
        lax.bitcast_convert_type(art_ref[...], jnp.float32),
    ], axis=0)
    fused_ref[:, 0:14] = stacked.T


def _tr_aemb_body(aembT_ref, aemb_out_ref):
    aemb_out_ref[...] = aembT_ref[...].T


def _transpose_tables(cont_feat_mapping, cat_map, art_map, artist_emb):
    contT = cont_feat_mapping.T
    catT = cat_map.T
    aembT = artist_emb.T
    W = 8192
    g1 = -(-N_TRACKS // W)
    fused = pl.pallas_call(
        _tr_fuse_body,
        grid=(g1,),
        in_specs=[
            pl.BlockSpec((N_CONT, W), lambda i: (0, i)),
            pl.BlockSpec((N_CAT, W), lambda i: (0, i)),
            pl.BlockSpec((1, W), lambda i: (0, i)),
        ],
        out_specs=pl.BlockSpec((W, 16), lambda i: (i, 0)),
        out_shape=jax.ShapeDtypeStruct((N_TRACKS, 16), jnp.float32),
    )(contT, catT, art_map.reshape(1, N_TRACKS))
    g2 = -(-100000 // W)
    aemb_rm = pl.pallas_call(
        _tr_aemb_body,
        grid=(g2,),
        in_specs=[pl.BlockSpec((D_ARTIST, W), lambda i: (0, i))],
        out_specs=pl.BlockSpec((W, D_ARTIST), lambda i: (i, 0)),
        out_shape=jax.ShapeDtypeStruct((100000, D_ARTIST), jnp.float32),
    )(aembT)
    return fused, aemb_rm


GPB = 4
BT = GPB * GTOK


def _tc_dense_body(cont_ref, art_ref, cat_ref, W1_ref, b1_ref, W2_ref, b2_ref,
                   pW_ref, pb_ref, g_ref, bb_ref, out_ref):
    prec = lax.Precision.DEFAULT
    dn_t = (((0,), (0,)), ((), ()))
    pW = pW_ref[...]
    base = D_ARTIST + D_CONT
    P2 = pW[D_ARTIST:base, :]
    W2P = jnp.dot(W2_ref[...], P2, precision=prec,
                  preferred_element_type=jnp.float32)
    bfold = pb_ref[...] + jnp.dot(b2_ref[...], P2, precision=prec,
                                  preferred_element_type=jnp.float32)
    cont9 = cont_ref[0:N_CONT, :]
    h = jnp.maximum(
        lax.dot_general(cont9, W1_ref[...], dn_t, precision=prec,
                        preferred_element_type=jnp.float32) + b1_ref[...], 0.0)
    cat32 = cat_ref[...].reshape(N_CAT * D_PER_CAT, BT)
    y = lax.dot_general(cat32, pW[base:, :], dn_t, precision=prec,
                        preferred_element_type=jnp.float32)
    y = y + jnp.dot(art_ref[...], pW[0:D_ARTIST, :], precision=prec,
                    preferred_element_type=jnp.float32)
    y = y + jnp.dot(h, W2P, precision=prec,
                    preferred_element_type=jnp.float32)
    y = y + bfold
    mu = jnp.mean(y, axis=-1, keepdims=True)
    d = y - mu
    var = jnp.mean(d * d, axis=-1, keepdims=True)
    out_ref[...] = d * lax.rsqrt(var + 1e-5) * g_ref[...] + bb_ref[...]


def kernel(x, cont_feat_mapping, cat_feat_mapping, artist_mapping, artist_emb,
           W1, b1, W2, b2, cat0, cat1, cat2, cat3, projW, projb, ln_g, ln_b):
    x3d = x.astype(jnp.int32).reshape(NGRP, GR, 128)
    fused, aemb_rm = _transpose_tables(
        cont_feat_mapping, cat_feat_mapping.astype(jnp.int32),
        artist_mapping.astype(jnp.int32), artist_emb)

    cont_g, art_g, cat_g = _sc_gather_call(
        x3d, fused, aemb_rm, cat0, cat1, cat2, cat3)

    art_g = art_g.reshape(NTOK, D_ARTIST)

    out = pl.pallas_call(
        _tc_dense_body,
        grid=(NTOK // BT,),
        in_specs=[
            pl.BlockSpec((16, BT), lambda i: (0, i)),
            pl.BlockSpec((BT, D_ARTIST), lambda i: (i, 0)),
            pl.BlockSpec((N_CAT, D_PER_CAT, BT), lambda i: (0, 0, i)),
            pl.BlockSpec((N_CONT, D_CONT), lambda i: (0, 0)),
            pl.BlockSpec((1, D_CONT), lambda i: (0, 0)),
            pl.BlockSpec((D_CONT, D_CONT), lambda i: (0, 0)),
            pl.BlockSpec((1, D_CONT), lambda i: (0, 0)),
            pl.BlockSpec((D_MODEL, D_MODEL), lambda i: (0, 0)),
            pl.BlockSpec((1, D_MODEL), lambda i: (0, 0)),
            pl.BlockSpec((1, D_MODEL), lambda i: (0, 0)),
            pl.BlockSpec((1, D_MODEL), lambda i: (0, 0)),
        ],
        out_specs=pl.BlockSpec((BT, D_MODEL), lambda i: (i, 0)),
        out_shape=jax.ShapeDtypeStruct((NTOK, D_MODEL), jnp.float32),
        compiler_params=pltpu.CompilerParams(
            fuse_transposed_lhs_in_matmul=True),
    )(cont_g, art_g, cat_g, W1, b1.reshape(1, -1), W2, b2.reshape(1, -1),
      projW, projb.reshape(1, -1), ln_g.reshape(1, -1), ln_b.reshape(1, -1))

    return out.reshape(B, T, D_MODEL)
